# baseline (speedup 1.0000x reference)
import functools

import jax
import jax.numpy as jnp
from jax import lax
from jax.experimental import pallas as pl
from jax.experimental.pallas import tpu as pltpu
from jax.experimental.pallas import tpu_sc as plsc

N = 10000
E = 320000
D = 128
G = 64

NC = 2
NS = 16
SS = 160
EPAD = SS * NS * 128
NP = 10240
NACC = NP
ZPW = NACC // NS

_mesh = plsc.VectorSubcoreMesh(core_axis_name="c", subcore_axis_name="s")


CH = 16
NCHUNK = SS // CH

@functools.partial(
    pl.kernel,
    out_type=[
        jax.ShapeDtypeStruct((NP, D), jnp.float32),
        jax.ShapeDtypeStruct((NP, D), jnp.float32),
    ],
    mesh=_mesh,
    scratch_types=[
        pltpu.VMEM((CH, 128), jnp.int32),
        pltpu.VMEM((CH, 128), jnp.int32),
        pltpu.VMEM((CH, 128), jnp.int32),
        pltpu.VMEM((CH, 128), jnp.int32),
        pltpu.VMEM((128, D), jnp.float32),
        pltpu.VMEM((128, D), jnp.float32),
        pltpu.VMEM_SHARED((NACC, D), jnp.float32),
        pltpu.SemaphoreType.DMA,
        pltpu.SemaphoreType.DMA,
        pltpu.SemaphoreType.DMA,
        pltpu.SemaphoreType.DMA,
        pltpu.SemaphoreType.DMA,
        pltpu.SemaphoreType.DMA,
    ],
)
def _msg_sc(yg_hbm, yh_hbm, srcg_hbm, dstg_hbm, srch_hbm, dsth_hbm,
            outg_hbm, outh_hbm,
            si0, di0, si1, di1, r0, r1, acc,
            g0, g1, s0, s1, i0, i1):
    c = lax.axis_index("c")
    s = lax.axis_index("s")
    rows = (r0, r1)
    sib = (si0, si1)
    dib = (di0, di1)
    gsem = (g0, g1)
    ssem = (s0, s1)
    isem = (i0, i1)

    def run(y_hbm, src3_hbm, dst3_hbm, out_hbm):
        pltpu.sync_copy(y_hbm.at[pl.ds(s * ZPW, ZPW)],
                        acc.at[pl.ds(s * ZPW, ZPW)])
        plsc.subcore_barrier()

        def fire_idx(cc, q):
            pltpu.async_copy(src3_hbm.at[s, pl.ds(cc * CH, CH)], sib[q],
                             isem[q])
            pltpu.async_copy(dst3_hbm.at[s, pl.ds(cc * CH, CH)], dib[q],
                             isem[q])

        def wait_idx(q):
            pltpu.make_async_copy(src3_hbm.at[s, pl.ds(0, CH)], sib[q],
                                  isem[q]).wait()
            pltpu.make_async_copy(dst3_hbm.at[s, pl.ds(0, CH)], dib[q],
                                  isem[q]).wait()

        def fire_g(q, r, b):
            pltpu.async_copy(y_hbm.at[sib[q].at[r, pl.ds(0, 64)]],
                             rows[b].at[pl.ds(0, 64)], gsem[b])
            pltpu.async_copy(y_hbm.at[sib[q].at[r, pl.ds(64, 64)]],
                             rows[b].at[pl.ds(64, 64)], gsem[b])

        def wait_g(q, b):
            pltpu.make_async_copy(y_hbm.at[sib[q].at[0, pl.ds(0, 64)]],
                                  rows[b].at[pl.ds(0, 64)], gsem[b]).wait()
            pltpu.make_async_copy(y_hbm.at[sib[q].at[0, pl.ds(64, 64)]],
                                  rows[b].at[pl.ds(64, 64)], gsem[b]).wait()

        def fire_s(q, r, b):
            pltpu.async_copy(rows[b], acc.at[dib[q].at[r]], ssem[b],
                             add=True)

        def wait_s(q, b):
            pltpu.make_async_copy(rows[b], acc.at[dib[q].at[0]],
                                  ssem[b]).wait()

        def half(q, r, b, skip_wait_s=False, next_=None):
            if not skip_wait_s:
                wait_s(q, b ^ 1)
            if next_ is not None:
                nq, nr = next_
                fire_g(nq, nr, b ^ 1)
            wait_g(q, b)
            fire_s(q, r, b)

        fire_idx(0, 0)
        wait_idx(0)
        fire_idx(1, 1)
        fire_g(0, 0, 0)

        for cc in range(NCHUNK):
            q = cc % 2
            npair = CH // 2

            def pair(rp, first=False, q=q):
                half(q, 2 * rp, 0, skip_wait_s=first, next_=(q, 2 * rp + 1))
                half(q, 2 * rp + 1, 1, next_=(q, 2 * rp + 2))

            pair(0, first=(cc == 0))
            if 1 <= cc < NCHUNK - 1:
                fire_idx(cc + 1, q ^ 1)
            lax.fori_loop(1, npair - 1,
                          lambda rp, _, q=q: (pair(rp), 0)[1], 0)

            rp = npair - 1
            half(q, 2 * rp, 0, next_=(q, 2 * rp + 1))
            wait_s(q, 0)
            if cc < NCHUNK - 1:
                wait_idx(q ^ 1)
                fire_g(q ^ 1, 0, 0)
            wait_g(q, 1)
            fire_s(q, 2 * rp + 1, 1)
        wait_s(0, 1)

        plsc.subcore_barrier()
        pltpu.sync_copy(acc.at[pl.ds(s * ZPW, ZPW)],
                        out_hbm.at[pl.ds(s * ZPW, ZPW)])

    @pl.when(c == 0)
    def _():
        run(yg_hbm, srcg_hbm, dstg_hbm, outg_hbm)

    @pl.when(c == 1)
    def _():
        run(yh_hbm, srch_hbm, dsth_hbm, outh_hbm)


@functools.partial(
    pl.kernel,
    out_type=[
        jax.ShapeDtypeStruct((NP, D), jnp.float32),
        jax.ShapeDtypeStruct((NP, D), jnp.float32),
    ],
    mesh=_mesh,
    scratch_types=[
        pltpu.VMEM((CH, 128), jnp.int32),
        pltpu.VMEM((CH, 128), jnp.int32),
        pltpu.VMEM((128, D), jnp.float32),
        pltpu.VMEM_SHARED((NACC, D), jnp.float32),
        pltpu.SemaphoreType.DMA,
        pltpu.SemaphoreType.DMA,
        pltpu.SemaphoreType.DMA,
        pltpu.SemaphoreType.DMA,
        pltpu.SemaphoreType.DMA,
        pltpu.SemaphoreType.DMA,
    ],
)
def _deg_sc(dstg_hbm, dsth_hbm, ones_hbm, outg_hbm, outh_hbm,
            di0, di1, ones_v, acc, d0, d1, d2, d3, i0, i1):
    c = lax.axis_index("c")
    s = lax.axis_index("s")
    dib = (di0, di1)
    dsem = (d0, d1, d2, d3)
    isem = (i0, i1)

    def run(dst3_hbm, out_hbm):
        pltpu.sync_copy(ones_hbm, ones_v)
        for t in range(ZPW // 128):
            pltpu.sync_copy(ones_v, acc.at[pl.ds(s * ZPW + t * 128, 128)])
        plsc.subcore_barrier()

        def fire_idx(cc, q):
            pltpu.async_copy(dst3_hbm.at[s, pl.ds(cc * CH, CH)], dib[q],
                             isem[q])

        def wait_idx(q):
            pltpu.make_async_copy(dst3_hbm.at[s, pl.ds(0, CH)], dib[q],
                                  isem[q]).wait()

        def fire_d(j):
            q, r = (j // CH) % 2, j % CH
            pltpu.async_copy(ones_v, acc.at[dib[q].at[r]], dsem[j % 4],
                             add=True)

        def wait_d(j):
            q = (j // CH) % 2
            pltpu.make_async_copy(ones_v, acc.at[dib[q].at[0]],
                                  dsem[j % 4]).wait()

        fire_idx(0, 0)
        wait_idx(0)
        fire_idx(1, 1)
        for j in range(SS):
            cc, r = j // CH, j % CH
            if r == 0 and cc >= 2:
                wait_idx(cc % 2)
            if j >= 4:
                wait_d(j - 4)
            if r == 4 and 1 <= cc < NCHUNK - 1:
                fire_idx(cc + 1, (cc + 1) % 2)
            fire_d(j)
        for j in range(SS - 4, SS):
            wait_d(j)

        plsc.subcore_barrier()
        pltpu.sync_copy(acc.at[pl.ds(s * ZPW, ZPW)],
                        out_hbm.at[pl.ds(s * ZPW, ZPW)])

    @pl.when(c == 0)
    def _():
        run(dstg_hbm, outg_hbm)

    @pl.when(c == 1)
    def _():
        run(dsth_hbm, outh_hbm)


def _prep_body(x_ref, wpre_ref, bpre_ref, w0_ref, deg_ref,
               x0_ref, y0_ref, dv_ref):
    x0 = jnp.dot(x_ref[...], wpre_ref[...],
                 preferred_element_type=jnp.float32) + bpre_ref[...]
    deg = deg_ref[:, 0:1]
    dinv = lax.rsqrt(jnp.maximum(deg, 1.0))
    x0_ref[...] = x0
    dv_ref[...] = jnp.broadcast_to(dinv, (NP, 8))
    y0_ref[...] = jnp.dot(x0, w0_ref[...],
                          preferred_element_type=jnp.float32) * dinv


_prep_tc = pl.pallas_call(
    _prep_body,
    out_shape=[
        jax.ShapeDtypeStruct((NP, D), jnp.float32),
        jax.ShapeDtypeStruct((NP, D), jnp.float32),
        jax.ShapeDtypeStruct((NP, 8), jnp.float32),
    ],
)


def _post_body(has_res, has_next, *refs):
    i = 0
    acc_ref = refs[i]; i += 1
    dv_ref = refs[i]; i += 1
    b_ref = refs[i]; i += 1
    res_ref = None
    w_ref = None
    if has_res:
        res_ref = refs[i]; i += 1
    if has_next:
        w_ref = refs[i]; i += 1
    x_ref = refs[i]; i += 1
    y_ref = refs[i] if has_next else None

    dv = dv_ref[:, 0:1]
    g = acc_ref[...] * dv + b_ref[...]
    if has_res:
        g = g + res_ref[...]
    x = jnp.maximum(g, 0.0)
    x_ref[...] = x
    if has_next:
        y_ref[...] = jnp.dot(x, w_ref[...],
                             preferred_element_type=jnp.float32) * dv


def _make_post(has_res, has_next):
    outs = [jax.ShapeDtypeStruct((NP, D), jnp.float32)]
    if has_next:
        outs.append(jax.ShapeDtypeStruct((NP, D), jnp.float32))
    return pl.pallas_call(
        functools.partial(_post_body, has_res, has_next),
        out_shape=outs,
    )


_post0 = _make_post(False, True)
_post1 = _make_post(True, True)
_post2 = _make_post(False, False)


def _pool_body(x0_ref, x1_ref, x2_ref, x3_ref, batch_ref,
               wp1_ref, bp1_ref, wp2_ref, bp2_ref, out_ref):
    iota = lax.broadcasted_iota(jnp.int32, (G, N), 0)
    oh = (batch_ref[...] == iota).astype(jnp.float32)
    cnt = jnp.dot(oh, jnp.ones((N, 1), jnp.float32),
                  preferred_element_type=jnp.float32)
    rc = 1.0 / jnp.maximum(cnt, 1.0)
    hpre = bp1_ref[...]
    for k, xr in enumerate((x0_ref, x1_ref, x2_ref, x3_ref)):
        pooled = jnp.dot(oh, xr[0:N, :],
                         preferred_element_type=jnp.float32) * rc
        hpre = hpre + jnp.dot(pooled, wp1_ref[k * D:(k + 1) * D, :],
                              preferred_element_type=jnp.float32)
    h = jnp.maximum(hpre, 0.0)
    out_ref[...] = jnp.dot(h, wp2_ref[...],
                           preferred_element_type=jnp.float32) + bp2_ref[...]


_pool_tc = pl.pallas_call(
    _pool_body,
    out_shape=jax.ShapeDtypeStruct((G, D), jnp.float32),
)


def _final_body(g_ref, h_ref, o_ref):
    o_ref[...] = jnp.sum(jnp.abs(g_ref[...] - h_ref[...]),
                         axis=1, keepdims=True)


_final_tc = pl.pallas_call(
    _final_body,
    out_shape=jax.ShapeDtypeStruct((G, 1), jnp.float32),
)


def _prep_edges(edge_index):
    src = edge_index[0].astype(jnp.int32)
    dst = edge_index[1].astype(jnp.int32)
    pad = EPAD - E
    src = jnp.concatenate([src, jnp.zeros((pad,), jnp.int32)])
    dst = jnp.concatenate([dst, jnp.full((pad,), N, jnp.int32)])
    return src.reshape(NS, SS, 128), dst.reshape(NS, SS, 128)


def _pad_x(x):
    return jnp.concatenate([x, jnp.zeros((NP - N, D), x.dtype)], axis=0)


def kernel(x_g, x_h, W_pre, b_pre, W0, b0, W1, b1, W2, b2, Wp1, bp1, Wp2, bp2,
           edge_index_g, edge_index_h, batch_g, batch_h):
    b_pre2 = b_pre.reshape(1, D)
    bs = (b0.reshape(1, D), b1.reshape(1, D), b2.reshape(1, D))
    bp12 = bp1.reshape(1, D)
    bp22 = bp2.reshape(1, D)

    srcg, dstg = _prep_edges(edge_index_g)
    srch, dsth = _prep_edges(edge_index_h)
    bg = batch_g.astype(jnp.int32).reshape(1, N)
    bh = batch_h.astype(jnp.int32).reshape(1, N)

    ones128 = jnp.ones((128, D), jnp.float32)
    degg, degh = _deg_sc(dstg, dsth, ones128)

    x0g, yg, dvg = _prep_tc(_pad_x(x_g), W_pre, b_pre2, W0, degg)
    x0h, yh, dvh = _prep_tc(_pad_x(x_h), W_pre, b_pre2, W0, degh)

    accg, acch = _msg_sc(yg, yh, srcg, dstg, srch, dsth)
    x1g, yg = _post0(accg, dvg, bs[0], W1)
    x1h, yh = _post0(acch, dvh, bs[0], W1)

    accg, acch = _msg_sc(yg, yh, srcg, dstg, srch, dsth)
    x2g, yg = _post1(accg, dvg, bs[1], x0g, W2)
    x2h, yh = _post1(acch, dvh, bs[1], x0h, W2)

    accg, acch = _msg_sc(yg, yh, srcg, dstg, srch, dsth)
    (x3g,) = _post2(accg, dvg, bs[2])
    (x3h,) = _post2(acch, dvh, bs[2])

    gg = _pool_tc(x0g, x1g, x2g, x3g, bg, Wp1, bp12, Wp2, bp22)
    hh = _pool_tc(x0h, x1h, x2h, x3h, bh, Wp1, bp12, Wp2, bp22)
    return _final_tc(gg, hh).reshape(G)


# speedup vs baseline: 1.0343x; 1.0343x over previous
import functools

import jax
import jax.numpy as jnp
from jax import lax
from jax.experimental import pallas as pl
from jax.experimental.pallas import tpu as pltpu
from jax.experimental.pallas import tpu_sc as plsc

N = 10000
E = 320000
D = 128
G = 64

NC = 2
NS = 16
SS = 160
EPAD = SS * NS * 128
NP = 10240
NACC = NP
ZPW = NACC // NS

_mesh = plsc.VectorSubcoreMesh(core_axis_name="c", subcore_axis_name="s")


CH = 16
NCHUNK = SS // CH

@functools.partial(
    pl.kernel,
    out_type=[
        jax.ShapeDtypeStruct((NP, D), jnp.float32),
        jax.ShapeDtypeStruct((NP, D), jnp.float32),
    ],
    mesh=_mesh,
    scratch_types=[
        pltpu.VMEM((CH, 128), jnp.int32),
        pltpu.VMEM((CH, 128), jnp.int32),
        pltpu.VMEM((CH, 128), jnp.int32),
        pltpu.VMEM((CH, 128), jnp.int32),
        pltpu.VMEM((128, D), jnp.float32),
        pltpu.VMEM((128, D), jnp.float32),
        pltpu.VMEM_SHARED((NACC, D), jnp.float32),
        pltpu.SemaphoreType.DMA,
        pltpu.SemaphoreType.DMA,
        pltpu.SemaphoreType.DMA,
        pltpu.SemaphoreType.DMA,
        pltpu.SemaphoreType.DMA,
        pltpu.SemaphoreType.DMA,
    ],
)
def _msg_sc(yg_hbm, yh_hbm, srcg_hbm, dstg_hbm, srch_hbm, dsth_hbm,
            outg_hbm, outh_hbm,
            si0, di0, si1, di1, r0, r1, acc,
            g0, g1, s0, s1, i0, i1):
    c = lax.axis_index("c")
    s = lax.axis_index("s")
    rows = (r0, r1)
    sib = (si0, si1)
    dib = (di0, di1)
    gsem = (g0, g1)
    ssem = (s0, s1)
    isem = (i0, i1)

    def run(y_hbm, src3_hbm, dst3_hbm, out_hbm):
        pltpu.sync_copy(y_hbm.at[pl.ds(s * ZPW, ZPW)],
                        acc.at[pl.ds(s * ZPW, ZPW)])
        plsc.subcore_barrier()

        def fire_idx(cc, q):
            pltpu.async_copy(src3_hbm.at[s, pl.ds(cc * CH, CH)], sib[q],
                             isem[q])
            pltpu.async_copy(dst3_hbm.at[s, pl.ds(cc * CH, CH)], dib[q],
                             isem[q])

        def wait_idx(q):
            pltpu.make_async_copy(src3_hbm.at[s, pl.ds(0, CH)], sib[q],
                                  isem[q]).wait()
            pltpu.make_async_copy(dst3_hbm.at[s, pl.ds(0, CH)], dib[q],
                                  isem[q]).wait()

        def fire_g(q, r, b):
            pltpu.async_copy(y_hbm.at[sib[q].at[r]], rows[b], gsem[b])

        def wait_g(q, b):
            pltpu.make_async_copy(y_hbm.at[sib[q].at[0]], rows[b],
                                  gsem[b]).wait()

        def fire_s(q, r, b):
            pltpu.async_copy(rows[b], acc.at[dib[q].at[r]], ssem[b],
                             add=True)

        def wait_s(q, b):
            pltpu.make_async_copy(rows[b], acc.at[dib[q].at[0]],
                                  ssem[b]).wait()

        fire_idx(0, 0)
        wait_idx(0)
        fire_idx(1, 1)
        fire_g(0, 0, 0)

        for j in range(SS):
            cc, r = j // CH, j % CH
            q, bb = cc % 2, j % 2
            if j > 0:
                wait_s(q, bb ^ 1)
            if r == 1 and 2 <= cc + 1 < NCHUNK:
                fire_idx(cc + 1, (cc + 1) % 2)
            jn = j + 1
            if jn < SS:
                qn, rn = (jn // CH) % 2, jn % CH
                if rn == 0:
                    wait_idx(qn)
                fire_g(qn, rn, bb ^ 1)
            wait_g(q, bb)
            fire_s(q, r, bb)
        wait_s((SS // CH - 1) % 2, (SS - 1) % 2)

        plsc.subcore_barrier()
        pltpu.sync_copy(acc.at[pl.ds(s * ZPW, ZPW)],
                        out_hbm.at[pl.ds(s * ZPW, ZPW)])

    @pl.when(c == 0)
    def _():
        run(yg_hbm, srcg_hbm, dstg_hbm, outg_hbm)

    @pl.when(c == 1)
    def _():
        run(yh_hbm, srch_hbm, dsth_hbm, outh_hbm)


@functools.partial(
    pl.kernel,
    out_type=[
        jax.ShapeDtypeStruct((NP, D), jnp.float32),
        jax.ShapeDtypeStruct((NP, D), jnp.float32),
    ],
    mesh=_mesh,
    scratch_types=[
        pltpu.VMEM((CH, 128), jnp.int32),
        pltpu.VMEM((CH, 128), jnp.int32),
        pltpu.VMEM((128, D), jnp.float32),
        pltpu.VMEM_SHARED((NACC, D), jnp.float32),
        pltpu.SemaphoreType.DMA,
        pltpu.SemaphoreType.DMA,
        pltpu.SemaphoreType.DMA,
        pltpu.SemaphoreType.DMA,
        pltpu.SemaphoreType.DMA,
        pltpu.SemaphoreType.DMA,
    ],
)
def _deg_sc(dstg_hbm, dsth_hbm, ones_hbm, outg_hbm, outh_hbm,
            di0, di1, ones_v, acc, d0, d1, d2, d3, i0, i1):
    c = lax.axis_index("c")
    s = lax.axis_index("s")
    dib = (di0, di1)
    dsem = (d0, d1, d2, d3)
    isem = (i0, i1)

    def run(dst3_hbm, out_hbm):
        pltpu.sync_copy(ones_hbm, ones_v)
        for t in range(ZPW // 128):
            pltpu.sync_copy(ones_v, acc.at[pl.ds(s * ZPW + t * 128, 128)])
        plsc.subcore_barrier()

        def fire_idx(cc, q):
            pltpu.async_copy(dst3_hbm.at[s, pl.ds(cc * CH, CH)], dib[q],
                             isem[q])

        def wait_idx(q):
            pltpu.make_async_copy(dst3_hbm.at[s, pl.ds(0, CH)], dib[q],
                                  isem[q]).wait()

        def fire_d(j):
            q, r = (j // CH) % 2, j % CH
            pltpu.async_copy(ones_v, acc.at[dib[q].at[r]], dsem[j % 4],
                             add=True)

        def wait_d(j):
            q = (j // CH) % 2
            pltpu.make_async_copy(ones_v, acc.at[dib[q].at[0]],
                                  dsem[j % 4]).wait()

        fire_idx(0, 0)
        wait_idx(0)
        fire_idx(1, 1)
        for j in range(SS):
            cc, r = j // CH, j % CH
            if r == 0 and cc >= 2:
                wait_idx(cc % 2)
            if j >= 4:
                wait_d(j - 4)
            if r == 4 and 1 <= cc < NCHUNK - 1:
                fire_idx(cc + 1, (cc + 1) % 2)
            fire_d(j)
        for j in range(SS - 4, SS):
            wait_d(j)

        plsc.subcore_barrier()
        pltpu.sync_copy(acc.at[pl.ds(s * ZPW, ZPW)],
                        out_hbm.at[pl.ds(s * ZPW, ZPW)])

    @pl.when(c == 0)
    def _():
        run(dstg_hbm, outg_hbm)

    @pl.when(c == 1)
    def _():
        run(dsth_hbm, outh_hbm)


def _prep_body(x_ref, wpre_ref, bpre_ref, w0_ref, deg_ref,
               x0_ref, y0_ref, dv_ref):
    x0 = jnp.dot(x_ref[...], wpre_ref[...],
                 preferred_element_type=jnp.float32) + bpre_ref[...]
    deg = deg_ref[:, 0:1]
    dinv = lax.rsqrt(jnp.maximum(deg, 1.0))
    x0_ref[...] = x0
    dv_ref[...] = jnp.broadcast_to(dinv, (NP, 8))
    y0_ref[...] = jnp.dot(x0, w0_ref[...],
                          preferred_element_type=jnp.float32) * dinv


_prep_tc = pl.pallas_call(
    _prep_body,
    out_shape=[
        jax.ShapeDtypeStruct((NP, D), jnp.float32),
        jax.ShapeDtypeStruct((NP, D), jnp.float32),
        jax.ShapeDtypeStruct((NP, 8), jnp.float32),
    ],
)


def _post_body(has_res, has_next, *refs):
    i = 0
    acc_ref = refs[i]; i += 1
    dv_ref = refs[i]; i += 1
    b_ref = refs[i]; i += 1
    res_ref = None
    w_ref = None
    if has_res:
        res_ref = refs[i]; i += 1
    if has_next:
        w_ref = refs[i]; i += 1
    x_ref = refs[i]; i += 1
    y_ref = refs[i] if has_next else None

    dv = dv_ref[:, 0:1]
    g = acc_ref[...] * dv + b_ref[...]
    if has_res:
        g = g + res_ref[...]
    x = jnp.maximum(g, 0.0)
    x_ref[...] = x
    if has_next:
        y_ref[...] = jnp.dot(x, w_ref[...],
                             preferred_element_type=jnp.float32) * dv


def _make_post(has_res, has_next):
    outs = [jax.ShapeDtypeStruct((NP, D), jnp.float32)]
    if has_next:
        outs.append(jax.ShapeDtypeStruct((NP, D), jnp.float32))
    return pl.pallas_call(
        functools.partial(_post_body, has_res, has_next),
        out_shape=outs,
    )


_post0 = _make_post(False, True)
_post1 = _make_post(True, True)
_post2 = _make_post(False, False)


def _pool_body(x0_ref, x1_ref, x2_ref, x3_ref, batch_ref,
               wp1_ref, bp1_ref, wp2_ref, bp2_ref, out_ref):
    iota = lax.broadcasted_iota(jnp.int32, (G, N), 0)
    oh = (batch_ref[...] == iota).astype(jnp.float32)
    cnt = jnp.dot(oh, jnp.ones((N, 1), jnp.float32),
                  preferred_element_type=jnp.float32)
    rc = 1.0 / jnp.maximum(cnt, 1.0)
    hpre = bp1_ref[...]
    for k, xr in enumerate((x0_ref, x1_ref, x2_ref, x3_ref)):
        pooled = jnp.dot(oh, xr[0:N, :],
                         preferred_element_type=jnp.float32) * rc
        hpre = hpre + jnp.dot(pooled, wp1_ref[k * D:(k + 1) * D, :],
                              preferred_element_type=jnp.float32)
    h = jnp.maximum(hpre, 0.0)
    out_ref[...] = jnp.dot(h, wp2_ref[...],
                           preferred_element_type=jnp.float32) + bp2_ref[...]


_pool_tc = pl.pallas_call(
    _pool_body,
    out_shape=jax.ShapeDtypeStruct((G, D), jnp.float32),
)


def _final_body(g_ref, h_ref, o_ref):
    o_ref[...] = jnp.sum(jnp.abs(g_ref[...] - h_ref[...]),
                         axis=1, keepdims=True)


_final_tc = pl.pallas_call(
    _final_body,
    out_shape=jax.ShapeDtypeStruct((G, 1), jnp.float32),
)


def _prep_edges(edge_index):
    src = edge_index[0].astype(jnp.int32)
    dst = edge_index[1].astype(jnp.int32)
    pad = EPAD - E
    src = jnp.concatenate([src, jnp.zeros((pad,), jnp.int32)])
    dst = jnp.concatenate([dst, jnp.full((pad,), N, jnp.int32)])
    return src.reshape(NS, SS, 128), dst.reshape(NS, SS, 128)


def _pad_x(x):
    return jnp.concatenate([x, jnp.zeros((NP - N, D), x.dtype)], axis=0)


def kernel(x_g, x_h, W_pre, b_pre, W0, b0, W1, b1, W2, b2, Wp1, bp1, Wp2, bp2,
           edge_index_g, edge_index_h, batch_g, batch_h):
    b_pre2 = b_pre.reshape(1, D)
    bs = (b0.reshape(1, D), b1.reshape(1, D), b2.reshape(1, D))
    bp12 = bp1.reshape(1, D)
    bp22 = bp2.reshape(1, D)

    srcg, dstg = _prep_edges(edge_index_g)
    srch, dsth = _prep_edges(edge_index_h)
    bg = batch_g.astype(jnp.int32).reshape(1, N)
    bh = batch_h.astype(jnp.int32).reshape(1, N)

    ones128 = jnp.ones((128, D), jnp.float32)
    degg, degh = _deg_sc(dstg, dsth, ones128)

    x0g, yg, dvg = _prep_tc(_pad_x(x_g), W_pre, b_pre2, W0, degg)
    x0h, yh, dvh = _prep_tc(_pad_x(x_h), W_pre, b_pre2, W0, degh)

    accg, acch = _msg_sc(yg, yh, srcg, dstg, srch, dsth)
    x1g, yg = _post0(accg, dvg, bs[0], W1)
    x1h, yh = _post0(acch, dvh, bs[0], W1)

    accg, acch = _msg_sc(yg, yh, srcg, dstg, srch, dsth)
    x2g, yg = _post1(accg, dvg, bs[1], x0g, W2)
    x2h, yh = _post1(acch, dvh, bs[1], x0h, W2)

    accg, acch = _msg_sc(yg, yh, srcg, dstg, srch, dsth)
    (x3g,) = _post2(accg, dvg, bs[2])
    (x3h,) = _post2(acch, dvh, bs[2])

    gg = _pool_tc(x0g, x1g, x2g, x3g, bg, Wp1, bp12, Wp2, bp22)
    hh = _pool_tc(x0h, x1h, x2h, x3h, bh, Wp1, bp12, Wp2, bp22)
    return _final_tc(gg, hh).reshape(G)


# speedup vs baseline: 1.0384x; 1.0040x over previous
import functools

import jax
import jax.numpy as jnp
from jax import lax
from jax.experimental import pallas as pl
from jax.experimental.pallas import tpu as pltpu
from jax.experimental.pallas import tpu_sc as plsc

N = 10000
E = 320000
D = 128
G = 64

NC = 2
NS = 16
SS = 160
EPAD = SS * NS * 128
NP = 10240
NACC = NP
ZPW = NACC // NS

_mesh = plsc.VectorSubcoreMesh(core_axis_name="c", subcore_axis_name="s")


CH = 16
NCHUNK = SS // CH

@functools.partial(
    pl.kernel,
    out_type=[
        jax.ShapeDtypeStruct((NP, D), jnp.float32),
        jax.ShapeDtypeStruct((NP, D), jnp.float32),
    ],
    mesh=_mesh,
    scratch_types=[
        pltpu.VMEM((CH, 128), jnp.int32),
        pltpu.VMEM((CH, 128), jnp.int32),
        pltpu.VMEM((CH, 128), jnp.int32),
        pltpu.VMEM((CH, 128), jnp.int32),
        pltpu.VMEM((128, D), jnp.float32),
        pltpu.VMEM((128, D), jnp.float32),
        pltpu.VMEM_SHARED((NACC, D), jnp.float32),
        pltpu.SemaphoreType.DMA,
        pltpu.SemaphoreType.DMA,
        pltpu.SemaphoreType.DMA,
        pltpu.SemaphoreType.DMA,
        pltpu.SemaphoreType.DMA,
        pltpu.SemaphoreType.DMA,
    ],
)
def _msg_sc(yg_hbm, yh_hbm, srcg_hbm, dstg_hbm, srch_hbm, dsth_hbm,
            outg_hbm, outh_hbm,
            si0, di0, si1, di1, r0, r1, acc,
            g0, g1, s0, s1, i0, i1):
    c = lax.axis_index("c")
    s = lax.axis_index("s")
    rows = (r0, r1)
    sib = (si0, si1)
    dib = (di0, di1)
    gsem = (g0, g1)
    ssem = (s0, s1)
    isem = (i0, i1)

    def run(y_hbm, src3_hbm, dst3_hbm, out_hbm):
        pltpu.sync_copy(y_hbm.at[pl.ds(s * ZPW, ZPW)],
                        acc.at[pl.ds(s * ZPW, ZPW)])
        plsc.subcore_barrier()

        def fire_idx(cc, q):
            pltpu.async_copy(src3_hbm.at[s, pl.ds(cc * CH, CH)], sib[q],
                             isem[q])
            pltpu.async_copy(dst3_hbm.at[s, pl.ds(cc * CH, CH)], dib[q],
                             isem[q])

        def wait_idx(q):
            pltpu.make_async_copy(src3_hbm.at[s, pl.ds(0, CH)], sib[q],
                                  isem[q]).wait()
            pltpu.make_async_copy(dst3_hbm.at[s, pl.ds(0, CH)], dib[q],
                                  isem[q]).wait()

        def fire_g(q, r, b):
            pltpu.async_copy(y_hbm.at[sib[q].at[r]], rows[b], gsem[b])

        def wait_g(q, b):
            pltpu.make_async_copy(y_hbm.at[sib[q].at[0]], rows[b],
                                  gsem[b]).wait()

        def fire_s(q, r, b):
            pltpu.async_copy(rows[b], acc.at[dib[q].at[r]], ssem[b],
                             add=True)

        def wait_s(q, b):
            pltpu.make_async_copy(rows[b], acc.at[dib[q].at[0]],
                                  ssem[b]).wait()

        fire_idx(0, 0)
        wait_idx(0)
        fire_idx(1, 1)
        fire_g(0, 0, 0)

        for j in range(SS):
            cc, r = j // CH, j % CH
            q, bb = cc % 2, j % 2
            if j > 0:
                wait_s(q, bb ^ 1)
            if r == 1 and 2 <= cc + 1 < NCHUNK:
                fire_idx(cc + 1, (cc + 1) % 2)
            jn = j + 1
            if jn < SS:
                qn, rn = (jn // CH) % 2, jn % CH
                if rn == 0:
                    wait_idx(qn)
                fire_g(qn, rn, bb ^ 1)
            wait_g(q, bb)
            fire_s(q, r, bb)
        wait_s((SS // CH - 1) % 2, (SS - 1) % 2)

        plsc.subcore_barrier()
        pltpu.sync_copy(acc.at[pl.ds(s * ZPW, ZPW)],
                        out_hbm.at[pl.ds(s * ZPW, ZPW)])

    @pl.when(c == 0)
    def _():
        run(yg_hbm, srcg_hbm, dstg_hbm, outg_hbm)

    @pl.when(c == 1)
    def _():
        run(yh_hbm, srch_hbm, dsth_hbm, outh_hbm)


@functools.partial(
    pl.kernel,
    out_type=[
        jax.ShapeDtypeStruct((NP, D), jnp.float32),
        jax.ShapeDtypeStruct((NP, D), jnp.float32),
    ],
    mesh=_mesh,
    scratch_types=[
        pltpu.VMEM((CH, 128), jnp.int32),
        pltpu.VMEM((CH, 128), jnp.int32),
        pltpu.VMEM((128, D), jnp.float32),
        pltpu.VMEM_SHARED((NACC, D), jnp.float32),
        pltpu.SemaphoreType.DMA,
        pltpu.SemaphoreType.DMA,
        pltpu.SemaphoreType.DMA,
        pltpu.SemaphoreType.DMA,
        pltpu.SemaphoreType.DMA,
        pltpu.SemaphoreType.DMA,
    ],
)
def _deg_sc(dstg_hbm, dsth_hbm, ones_hbm, outg_hbm, outh_hbm,
            di0, di1, ones_v, acc, d0, d1, d2, d3, i0, i1):
    c = lax.axis_index("c")
    s = lax.axis_index("s")
    dib = (di0, di1)
    dsem = (d0, d1, d2, d3)
    isem = (i0, i1)

    def run(dst3_hbm, out_hbm):
        pltpu.sync_copy(ones_hbm, ones_v)
        for t in range(ZPW // 128):
            pltpu.sync_copy(ones_v, acc.at[pl.ds(s * ZPW + t * 128, 128)])
        plsc.subcore_barrier()

        def fire_idx(cc, q):
            pltpu.async_copy(dst3_hbm.at[s, pl.ds(cc * CH, CH)], dib[q],
                             isem[q])

        def wait_idx(q):
            pltpu.make_async_copy(dst3_hbm.at[s, pl.ds(0, CH)], dib[q],
                                  isem[q]).wait()

        def fire_d(j):
            q, r = (j // CH) % 2, j % CH
            pltpu.async_copy(ones_v, acc.at[dib[q].at[r]], dsem[j % 4],
                             add=True)

        def wait_d(j):
            q = (j // CH) % 2
            pltpu.make_async_copy(ones_v, acc.at[dib[q].at[0]],
                                  dsem[j % 4]).wait()

        fire_idx(0, 0)
        wait_idx(0)
        fire_idx(1, 1)
        for j in range(SS):
            cc, r = j // CH, j % CH
            if r == 0 and cc >= 1:
                wait_idx(cc % 2)
            if j >= 4:
                wait_d(j - 4)
            if r == 4 and 1 <= cc < NCHUNK - 1:
                fire_idx(cc + 1, (cc + 1) % 2)
            fire_d(j)
        for j in range(SS - 4, SS):
            wait_d(j)

        plsc.subcore_barrier()
        pltpu.sync_copy(acc.at[pl.ds(s * ZPW, ZPW)],
                        out_hbm.at[pl.ds(s * ZPW, ZPW)])

    @pl.when(c == 0)
    def _():
        run(dstg_hbm, outg_hbm)

    @pl.when(c == 1)
    def _():
        run(dsth_hbm, outh_hbm)


def _prep_body(x_ref, wpre_ref, bpre_ref, w0_ref, deg_ref,
               x0_ref, y0_ref, dv_ref):
    x0 = jnp.dot(x_ref[...], wpre_ref[...],
                 preferred_element_type=jnp.float32) + bpre_ref[...]
    deg = deg_ref[:, 0:1]
    dinv = lax.rsqrt(jnp.maximum(deg, 1.0))
    x0_ref[...] = x0
    dv_ref[...] = jnp.broadcast_to(dinv, (NP, 8))
    y0_ref[...] = jnp.dot(x0, w0_ref[...],
                          preferred_element_type=jnp.float32) * dinv


_prep_tc = pl.pallas_call(
    _prep_body,
    out_shape=[
        jax.ShapeDtypeStruct((NP, D), jnp.float32),
        jax.ShapeDtypeStruct((NP, D), jnp.float32),
        jax.ShapeDtypeStruct((NP, 8), jnp.float32),
    ],
)


def _post_body(has_res, has_next, *refs):
    i = 0
    acc_ref = refs[i]; i += 1
    dv_ref = refs[i]; i += 1
    b_ref = refs[i]; i += 1
    res_ref = None
    w_ref = None
    if has_res:
        res_ref = refs[i]; i += 1
    if has_next:
        w_ref = refs[i]; i += 1
    x_ref = refs[i]; i += 1
    y_ref = refs[i] if has_next else None

    dv = dv_ref[:, 0:1]
    g = acc_ref[...] * dv + b_ref[...]
    if has_res:
        g = g + res_ref[...]
    x = jnp.maximum(g, 0.0)
    x_ref[...] = x
    if has_next:
        y_ref[...] = jnp.dot(x, w_ref[...],
                             preferred_element_type=jnp.float32) * dv


def _make_post(has_res, has_next):
    outs = [jax.ShapeDtypeStruct((NP, D), jnp.float32)]
    if has_next:
        outs.append(jax.ShapeDtypeStruct((NP, D), jnp.float32))
    return pl.pallas_call(
        functools.partial(_post_body, has_res, has_next),
        out_shape=outs,
    )


_post0 = _make_post(False, True)
_post1 = _make_post(True, True)
_post2 = _make_post(False, False)


def _pool_body(x0_ref, x1_ref, x2_ref, x3_ref, batch_ref,
               wp1_ref, bp1_ref, wp2_ref, bp2_ref, out_ref):
    iota = lax.broadcasted_iota(jnp.int32, (G, N), 0)
    oh = (batch_ref[...] == iota).astype(jnp.float32)
    cnt = jnp.dot(oh, jnp.ones((N, 1), jnp.float32),
                  preferred_element_type=jnp.float32)
    rc = 1.0 / jnp.maximum(cnt, 1.0)
    hpre = bp1_ref[...]
    for k, xr in enumerate((x0_ref, x1_ref, x2_ref, x3_ref)):
        pooled = jnp.dot(oh, xr[0:N, :],
                         preferred_element_type=jnp.float32) * rc
        hpre = hpre + jnp.dot(pooled, wp1_ref[k * D:(k + 1) * D, :],
                              preferred_element_type=jnp.float32)
    h = jnp.maximum(hpre, 0.0)
    out_ref[...] = jnp.dot(h, wp2_ref[...],
                           preferred_element_type=jnp.float32) + bp2_ref[...]


_pool_tc = pl.pallas_call(
    _pool_body,
    out_shape=jax.ShapeDtypeStruct((G, D), jnp.float32),
)


def _final_body(g_ref, h_ref, o_ref):
    o_ref[...] = jnp.sum(jnp.abs(g_ref[...] - h_ref[...]),
                         axis=1, keepdims=True)


_final_tc = pl.pallas_call(
    _final_body,
    out_shape=jax.ShapeDtypeStruct((G, 1), jnp.float32),
)


def _prep_edges(edge_index):
    src = edge_index[0].astype(jnp.int32)
    dst = edge_index[1].astype(jnp.int32)
    pad = EPAD - E
    src = jnp.concatenate([src, jnp.zeros((pad,), jnp.int32)])
    dst = jnp.concatenate([dst, jnp.full((pad,), N, jnp.int32)])
    return src.reshape(NS, SS, 128), dst.reshape(NS, SS, 128)


def _pad_x(x):
    return jnp.concatenate([x, jnp.zeros((NP - N, D), x.dtype)], axis=0)


def kernel(x_g, x_h, W_pre, b_pre, W0, b0, W1, b1, W2, b2, Wp1, bp1, Wp2, bp2,
           edge_index_g, edge_index_h, batch_g, batch_h):
    b_pre2 = b_pre.reshape(1, D)
    bs = (b0.reshape(1, D), b1.reshape(1, D), b2.reshape(1, D))
    bp12 = bp1.reshape(1, D)
    bp22 = bp2.reshape(1, D)

    srcg, dstg = _prep_edges(edge_index_g)
    srch, dsth = _prep_edges(edge_index_h)
    bg = batch_g.astype(jnp.int32).reshape(1, N)
    bh = batch_h.astype(jnp.int32).reshape(1, N)

    ones128 = jnp.ones((128, D), jnp.float32)
    degg, degh = _deg_sc(dstg, dsth, ones128)

    x0g, yg, dvg = _prep_tc(_pad_x(x_g), W_pre, b_pre2, W0, degg)
    x0h, yh, dvh = _prep_tc(_pad_x(x_h), W_pre, b_pre2, W0, degh)

    accg, acch = _msg_sc(yg, yh, srcg, dstg, srch, dsth)
    x1g, yg = _post0(accg, dvg, bs[0], W1)
    x1h, yh = _post0(acch, dvh, bs[0], W1)

    accg, acch = _msg_sc(yg, yh, srcg, dstg, srch, dsth)
    x2g, yg = _post1(accg, dvg, bs[1], x0g, W2)
    x2h, yh = _post1(acch, dvh, bs[1], x0h, W2)

    accg, acch = _msg_sc(yg, yh, srcg, dstg, srch, dsth)
    (x3g,) = _post2(accg, dvg, bs[2])
    (x3h,) = _post2(acch, dvh, bs[2])

    gg = _pool_tc(x0g, x1g, x2g, x3g, bg, Wp1, bp12, Wp2, bp22)
    hh = _pool_tc(x0h, x1h, x2h, x3h, bh, Wp1, bp12, Wp2, bp22)
    return _final_tc(gg, hh).reshape(G)
